# Initial kernel scaffold; baseline (speedup 1.0000x reference)
import jax
import jax.numpy as jnp
from jax.experimental import pallas as pl


def kernel(x, edge_index, eW1, eb1, eW2, eb2, eW3, eb3, dW1, db1, dW2, db2, dW3, db3, c1Wl, c1b, c1Wr, c2Wl, c2b, c2Wr, c3Wl, c3b, c3Wr, fcW, fcb):
    raise NotImplementedError("write your pallas kernel here")


# speedup vs baseline: 7.0807x; 7.0807x over previous
import functools

import jax
import jax.numpy as jnp
from jax import lax
from jax.experimental import pallas as pl
from jax.experimental.pallas import tpu as pltpu
from jax.experimental.pallas import tpu_sc as plsc

_N = 100000
_E = 1600000
_NPAD = 102400
_EPAD = 1638400
_K = 1024
_BN = 2000
_NTILES = 16
_STRIPE = _NPAD // _NTILES


def _full_spec(a):
    nd = a.ndim
    return pl.BlockSpec(a.shape, lambda i, _nd=nd: (0,) * _nd)


def _enc_body(x_ref, w1, b1, w2, b2, w3, b3, out_ref):
    h = jnp.maximum(jnp.dot(x_ref[...], w1[...], preferred_element_type=jnp.float32) + b1[...], 0.0)
    h = jnp.maximum(jnp.dot(h, w2[...], preferred_element_type=jnp.float32) + b2[...], 0.0)
    out_ref[...] = jnp.maximum(jnp.dot(h, w3[...], preferred_element_type=jnp.float32) + b3[...], 0.0)


def _tc_encoder(x, eW1, eb1, eW2, eb2, eW3, eb3):
    args = (x, eW1, eb1, eW2, eb2, eW3, eb3)
    return pl.pallas_call(
        _enc_body,
        grid=(_N // _BN,),
        in_specs=[pl.BlockSpec((_BN, 2), lambda i: (i, 0))] + [_full_spec(a) for a in args[1:]],
        out_specs=pl.BlockSpec((_BN, 32), lambda i: (i, 0)),
        out_shape=jax.ShapeDtypeStruct((_N, 32), jnp.float32),
    )(*args)


def _combine_body(agg_ref, inv_ref, prev_ref, wl, wr, b, out_ref):
    inv = inv_ref[...]
    a = agg_ref[...]
    s = jnp.concatenate([a[j] for j in range(a.shape[0])], axis=1) * inv
    out_ref[...] = jnp.maximum(
        jnp.dot(s, wl[...], preferred_element_type=jnp.float32)
        + jnp.dot(prev_ref[...], wr[...], preferred_element_type=jnp.float32)
        + b[...], 0.0)


def _tc_combine(agg, cnt, prev, wl, wr, b, out_dim):
    nh = agg.shape[0]
    args = (agg, cnt, prev, wl, wr, b)
    return pl.pallas_call(
        _combine_body,
        grid=(_N // _BN,),
        in_specs=[
            pl.BlockSpec((nh, _BN, 16), lambda i: (0, i, 0)),
            pl.BlockSpec((_BN, 1), lambda i: (i, 0)),
            pl.BlockSpec((_BN, prev.shape[1]), lambda i: (i, 0)),
            _full_spec(wl), _full_spec(wr), _full_spec(b),
        ],
        out_specs=pl.BlockSpec((_BN, out_dim), lambda i: (i, 0)),
        out_shape=jax.ShapeDtypeStruct((_N, out_dim), jnp.float32),
    )(*args)


def _combine2_body(agg_ref, inv_ref, prev_ref, wl, wr, b, w3l, w3r, p_ref, q_ref):
    inv = inv_ref[...]
    a = agg_ref[...]
    s = jnp.concatenate([a[j] for j in range(a.shape[0])], axis=1) * inv
    h2 = jnp.maximum(
        jnp.dot(s, wl[...], preferred_element_type=jnp.float32)
        + jnp.dot(prev_ref[...], wr[...], preferred_element_type=jnp.float32)
        + b[...], 0.0)
    p_ref[...] = jnp.dot(h2, w3l[...], preferred_element_type=jnp.float32)
    q_ref[...] = jnp.dot(h2, w3r[...], preferred_element_type=jnp.float32)


def _tc_combine2(agg, cnt, prev, wl, wr, b, w3l, w3r):
    args = (agg, cnt, prev, wl, wr, b, w3l, w3r)
    return pl.pallas_call(
        _combine2_body,
        grid=(_N // _BN,),
        in_specs=[
            pl.BlockSpec((4, _BN, 16), lambda i: (0, i, 0)),
            pl.BlockSpec((_BN, 1), lambda i: (i, 0)),
            pl.BlockSpec((_BN, 64), lambda i: (i, 0)),
            _full_spec(wl), _full_spec(wr), _full_spec(b),
            _full_spec(w3l), _full_spec(w3r),
        ],
        out_specs=[
            pl.BlockSpec((_BN, 32), lambda i: (i, 0)),
            pl.BlockSpec((_BN, 32), lambda i: (i, 0)),
        ],
        out_shape=[
            jax.ShapeDtypeStruct((_N, 32), jnp.float32),
            jax.ShapeDtypeStruct((_N, 32), jnp.float32),
        ],
    )(*args)


def _final_body(agg_ref, inv_ref, q_ref, b3, fcw, fcb, out_ref, acc_ref):
    i = pl.program_id(0)
    inv = inv_ref[...]
    a = agg_ref[...]
    h3 = jnp.maximum(
        jnp.concatenate([a[0], a[1]], axis=1) * inv + q_ref[...] + b3[...], 0.0)

    @pl.when(i == 0)
    def _():
        acc_ref[...] = jnp.zeros_like(acc_ref)

    acc_ref[...] = acc_ref[...] + jnp.sum(h3, axis=0, keepdims=True)

    @pl.when(i == pl.num_programs(0) - 1)
    def _():
        pooled = acc_ref[...] * (1.0 / _N)
        logits = jnp.dot(pooled, fcw[...], preferred_element_type=jnp.float32) + fcb[...]
        out_ref[...] = 1.0 / (1.0 + jnp.exp(-logits))


def _tc_final(agg, cnt, q, b3, fcW, fcb):
    args = (agg, cnt, q, b3, fcW, fcb)
    return pl.pallas_call(
        _final_body,
        grid=(_N // _BN,),
        in_specs=[
            pl.BlockSpec((2, _BN, 16), lambda i: (0, i, 0)),
            pl.BlockSpec((_BN, 1), lambda i: (i, 0)),
            pl.BlockSpec((_BN, 32), lambda i: (i, 0)),
            _full_spec(b3), _full_spec(fcW), _full_spec(fcb),
        ],
        out_specs=pl.BlockSpec((1, 10), lambda i: (0, 0)),
        out_shape=jax.ShapeDtypeStruct((1, 10), jnp.float32),
        scratch_shapes=[pltpu.VMEM((1, 32), jnp.float32)],
    )(*args)


def _sc_degree(dstp):
    ept = _EPAD // 32
    nch = ept // _K
    mesh = plsc.VectorSubcoreMesh(core_axis_name="c", subcore_axis_name="s")

    @functools.partial(
        pl.kernel, mesh=mesh,
        compiler_params=pltpu.CompilerParams(
            use_tc_tiling_on_sc=False, needs_layout_passes=False),
        out_type=jax.ShapeDtypeStruct((32 * _NPAD,), jnp.float32),
        scratch_types=[
            pltpu.VMEM((_NPAD,), jnp.float32),
            pltpu.VMEM((_K,), jnp.int32),
        ],
    )
    def k(dst_hbm, out_hbm, cnt_v, dst_v):
        c = lax.axis_index("c")
        s = lax.axis_index("s")
        w = s * 2 + c
        zero16 = jnp.zeros((16,), jnp.float32)

        def zbody(i, _):
            cnt_v[pl.ds(i * 16, 16)] = zero16
            return 0
        lax.fori_loop(0, _NPAD // 16, zbody, 0)

        base = w * ept
        ones = jnp.ones((16,), jnp.float32)

        def chunk(j, _):
            pltpu.sync_copy(dst_hbm.at[pl.ds(base + j * _K, _K)], dst_v)

            def inner(i, _):
                idx = dst_v[pl.ds(i * 16, 16)]
                plsc.addupdate_scatter(cnt_v, [idx], ones)
                return 0
            lax.fori_loop(0, _K // 16, inner, 0)
            return 0
        lax.fori_loop(0, nch, chunk, 0)
        pltpu.sync_copy(cnt_v, out_hbm.at[pl.ds(w * _NPAD, _NPAD)])

    return k(dstp)


def _sc_invdeg(parts):
    stripe = _NPAD // 32
    mesh = plsc.VectorSubcoreMesh(core_axis_name="c", subcore_axis_name="s")

    @functools.partial(
        pl.kernel, mesh=mesh,
        compiler_params=pltpu.CompilerParams(
            use_tc_tiling_on_sc=False, needs_layout_passes=False),
        out_type=jax.ShapeDtypeStruct((_NPAD,), jnp.float32),
        scratch_types=[
            pltpu.VMEM((stripe,), jnp.float32),
            pltpu.VMEM((stripe,), jnp.float32),
        ],
    )
    def k(part_hbm, out_hbm, red_v, tmp_v):
        c = lax.axis_index("c")
        s = lax.axis_index("s")
        w = s * 2 + c
        o = w * stripe
        pltpu.sync_copy(part_hbm.at[pl.ds(o, stripe)], red_v)
        for p in range(1, 32):
            pltpu.sync_copy(part_hbm.at[pl.ds(p * _NPAD + o, stripe)], tmp_v)

            def abody(i, _):
                sl = pl.ds(i * 16, 16)
                red_v[sl] = red_v[sl] + tmp_v[sl]
                return 0
            lax.fori_loop(0, stripe // 16, abody, 0)

        one16 = jnp.full((16,), 1.0, jnp.float32)

        def ibody(i, _):
            sl = pl.ds(i * 16, 16)
            red_v[sl] = one16 / jnp.maximum(red_v[sl], one16)
            return 0
        lax.fori_loop(0, stripe // 16, ibody, 0)
        pltpu.sync_copy(red_v, out_hbm.at[pl.ds(o, stripe)])

    return k(parts)


def _sc_segsum(table_flat, srcp, dstp, zeros16, nhalves):
    npass = nhalves // 2
    ept = _EPAD // _NTILES
    nch = ept // _K
    mesh = plsc.VectorSubcoreMesh(core_axis_name="c", subcore_axis_name="s")

    @functools.partial(
        pl.kernel, mesh=mesh,
        compiler_params=pltpu.CompilerParams(
            use_tc_tiling_on_sc=False, needs_layout_passes=False),
        out_type=jax.ShapeDtypeStruct((nhalves, _NPAD, 16), jnp.float32),
        scratch_types=[
            pltpu.VMEM((_K,), jnp.int32),
            pltpu.VMEM((_K,), jnp.int32),
            pltpu.VMEM((_K,), jnp.int32),
            pltpu.VMEM((_K, 16), jnp.float32),
            pltpu.VMEM_SHARED((_NPAD, 16), jnp.float32),
            pltpu.SemaphoreType.DMA,
        ],
    )
    def k(tab_hbm, src_hbm, dst_hbm, z_hbm, out_hbm,
          src_v, gidx_v, dst_v, rows_v, acc_sh, sem):
        c = lax.axis_index("c")
        s = lax.axis_index("s")
        r0 = s * _STRIPE
        base = s * ept

        for p in range(npass):
            h = c * npass + p
            pltpu.sync_copy(z_hbm.at[pl.ds(r0, _STRIPE)], acc_sh.at[pl.ds(r0, _STRIPE)])
            plsc.subcore_barrier()

            def chunk(j, _):
                e0 = base + j * _K
                pltpu.sync_copy(src_hbm.at[pl.ds(e0, _K)], src_v)
                pltpu.sync_copy(dst_hbm.at[pl.ds(e0, _K)], dst_v)

                def xform(i, _):
                    v = src_v[pl.ds(i * 16, 16)]
                    gidx_v[pl.ds(i * 16, 16)] = v * nhalves + h
                    return 0
                lax.fori_loop(0, _K // 16, xform, 0)
                pltpu.async_copy(tab_hbm.at[gidx_v], rows_v, sem).wait()
                pltpu.sync_copy(rows_v, acc_sh.at[dst_v], add=True)
                return 0
            lax.fori_loop(0, nch, chunk, 0)
            plsc.subcore_barrier()
            pltpu.sync_copy(acc_sh.at[pl.ds(r0, _STRIPE)],
                            out_hbm.at[h, pl.ds(r0, _STRIPE)])
            plsc.subcore_barrier()

    return k(table_flat, srcp, dstp, zeros16)


def kernel(x, edge_index, eW1, eb1, eW2, eb2, eW3, eb3, dW1, db1, dW2, db2,
           dW3, db3, c1Wl, c1b, c1Wr, c2Wl, c2b, c2Wr, c3Wl, c3b, c3Wr, fcW, fcb):
    del dW1, db1, dW2, db2, dW3, db3

    eb1r = eb1.reshape(1, 32)
    eb2r = eb2.reshape(1, 32)
    eb3r = eb3.reshape(1, 32)
    c1br = c1b.reshape(1, 64)
    c2br = c2b.reshape(1, 64)
    c3br = c3b.reshape(1, 32)
    fcbr = fcb.reshape(1, 10)

    src = edge_index[0]
    dst = edge_index[1]
    srcp = jnp.concatenate([src, jnp.zeros((_EPAD - _E,), jnp.int32)])
    dstp = jnp.concatenate([dst, jnp.full((_EPAD - _E,), _NPAD - 1, jnp.int32)])
    zeros16 = jnp.zeros((_NPAD, 16), jnp.float32)

    inv = _sc_invdeg(_sc_degree(dstp)).reshape(_NPAD, 1)

    encoded = _tc_encoder(x, eW1, eb1r, eW2, eb2r, eW3, eb3r)

    agg1 = _sc_segsum(encoded.reshape(2 * _N, 16), srcp, dstp, zeros16, 2)
    h1 = _tc_combine(agg1, inv, encoded, c1Wl, c1Wr, c1br, 64)

    agg2 = _sc_segsum(h1.reshape(4 * _N, 16), srcp, dstp, zeros16, 4)
    p3, q3 = _tc_combine2(agg2, inv, h1, c2Wl, c2Wr, c2br, c3Wl, c3Wr)

    agg3 = _sc_segsum(p3.reshape(2 * _N, 16), srcp, dstp, zeros16, 2)
    return _tc_final(agg3, inv, q3, c3br, fcW, fcbr)


# speedup vs baseline: 7.5669x; 1.0687x over previous
import functools

import jax
import jax.numpy as jnp
from jax import lax
from jax.experimental import pallas as pl
from jax.experimental.pallas import tpu as pltpu
from jax.experimental.pallas import tpu_sc as plsc

_N = 100000
_E = 1600000
_NPAD = 102400
_EPAD = 1638400
_K = 1024
_KS = 512
_BN = 2000
_NTILES = 16
_STRIPE = _NPAD // _NTILES


def _full_spec(a):
    nd = a.ndim
    return pl.BlockSpec(a.shape, lambda i, _nd=nd: (0,) * _nd)


def _enc_body(x_ref, w1, b1, w2, b2, w3, b3, out_ref):
    h = jnp.maximum(jnp.dot(x_ref[...], w1[...], preferred_element_type=jnp.float32) + b1[...], 0.0)
    h = jnp.maximum(jnp.dot(h, w2[...], preferred_element_type=jnp.float32) + b2[...], 0.0)
    out_ref[...] = jnp.maximum(jnp.dot(h, w3[...], preferred_element_type=jnp.float32) + b3[...], 0.0)


def _tc_encoder(x, eW1, eb1, eW2, eb2, eW3, eb3):
    args = (x, eW1, eb1, eW2, eb2, eW3, eb3)
    return pl.pallas_call(
        _enc_body,
        grid=(_N // _BN,),
        in_specs=[pl.BlockSpec((_BN, 2), lambda i: (i, 0))] + [_full_spec(a) for a in args[1:]],
        out_specs=pl.BlockSpec((_BN, 32), lambda i: (i, 0)),
        out_shape=jax.ShapeDtypeStruct((_N, 32), jnp.float32),
    )(*args)


def _combine_body(agg_ref, inv_ref, prev_ref, wl, wr, b, out_ref):
    inv = inv_ref[...]
    a = agg_ref[...]
    s = jnp.concatenate([a[j] for j in range(a.shape[0])], axis=1) * inv
    out_ref[...] = jnp.maximum(
        jnp.dot(s, wl[...], preferred_element_type=jnp.float32)
        + jnp.dot(prev_ref[...], wr[...], preferred_element_type=jnp.float32)
        + b[...], 0.0)


def _tc_combine(agg, cnt, prev, wl, wr, b, out_dim):
    nh = agg.shape[0]
    args = (agg, cnt, prev, wl, wr, b)
    return pl.pallas_call(
        _combine_body,
        grid=(_N // _BN,),
        in_specs=[
            pl.BlockSpec((nh, _BN, 16), lambda i: (0, i, 0)),
            pl.BlockSpec((_BN, 1), lambda i: (i, 0)),
            pl.BlockSpec((_BN, prev.shape[1]), lambda i: (i, 0)),
            _full_spec(wl), _full_spec(wr), _full_spec(b),
        ],
        out_specs=pl.BlockSpec((_BN, out_dim), lambda i: (i, 0)),
        out_shape=jax.ShapeDtypeStruct((_N, out_dim), jnp.float32),
    )(*args)


def _combine2_body(agg_ref, inv_ref, prev_ref, wl, wr, b, w3l, w3r, p_ref, q_ref):
    inv = inv_ref[...]
    a = agg_ref[...]
    s = jnp.concatenate([a[j] for j in range(a.shape[0])], axis=1) * inv
    h2 = jnp.maximum(
        jnp.dot(s, wl[...], preferred_element_type=jnp.float32)
        + jnp.dot(prev_ref[...], wr[...], preferred_element_type=jnp.float32)
        + b[...], 0.0)
    p_ref[...] = jnp.dot(h2, w3l[...], preferred_element_type=jnp.float32)
    q_ref[...] = jnp.dot(h2, w3r[...], preferred_element_type=jnp.float32)


def _tc_combine2(agg, cnt, prev, wl, wr, b, w3l, w3r):
    args = (agg, cnt, prev, wl, wr, b, w3l, w3r)
    return pl.pallas_call(
        _combine2_body,
        grid=(_N // _BN,),
        in_specs=[
            pl.BlockSpec((4, _BN, 16), lambda i: (0, i, 0)),
            pl.BlockSpec((_BN, 1), lambda i: (i, 0)),
            pl.BlockSpec((_BN, 64), lambda i: (i, 0)),
            _full_spec(wl), _full_spec(wr), _full_spec(b),
            _full_spec(w3l), _full_spec(w3r),
        ],
        out_specs=[
            pl.BlockSpec((_BN, 32), lambda i: (i, 0)),
            pl.BlockSpec((_BN, 32), lambda i: (i, 0)),
        ],
        out_shape=[
            jax.ShapeDtypeStruct((_N, 32), jnp.float32),
            jax.ShapeDtypeStruct((_N, 32), jnp.float32),
        ],
    )(*args)


def _final_body(agg_ref, inv_ref, q_ref, b3, fcw, fcb, out_ref, acc_ref):
    i = pl.program_id(0)
    inv = inv_ref[...]
    a = agg_ref[...]
    h3 = jnp.maximum(
        jnp.concatenate([a[0], a[1]], axis=1) * inv + q_ref[...] + b3[...], 0.0)

    @pl.when(i == 0)
    def _():
        acc_ref[...] = jnp.zeros_like(acc_ref)

    acc_ref[...] = acc_ref[...] + jnp.sum(h3, axis=0, keepdims=True)

    @pl.when(i == pl.num_programs(0) - 1)
    def _():
        pooled = acc_ref[...] * (1.0 / _N)
        logits = jnp.dot(pooled, fcw[...], preferred_element_type=jnp.float32) + fcb[...]
        out_ref[...] = 1.0 / (1.0 + jnp.exp(-logits))


def _tc_final(agg, cnt, q, b3, fcW, fcb):
    args = (agg, cnt, q, b3, fcW, fcb)
    return pl.pallas_call(
        _final_body,
        grid=(_N // _BN,),
        in_specs=[
            pl.BlockSpec((2, _BN, 16), lambda i: (0, i, 0)),
            pl.BlockSpec((_BN, 1), lambda i: (i, 0)),
            pl.BlockSpec((_BN, 32), lambda i: (i, 0)),
            _full_spec(b3), _full_spec(fcW), _full_spec(fcb),
        ],
        out_specs=pl.BlockSpec((1, 10), lambda i: (0, 0)),
        out_shape=jax.ShapeDtypeStruct((1, 10), jnp.float32),
        scratch_shapes=[pltpu.VMEM((1, 32), jnp.float32)],
    )(*args)


def _sc_degree(dstp):
    ept = _EPAD // 32
    nch = ept // _K
    mesh = plsc.VectorSubcoreMesh(core_axis_name="c", subcore_axis_name="s")

    @functools.partial(
        pl.kernel, mesh=mesh,
        compiler_params=pltpu.CompilerParams(
            use_tc_tiling_on_sc=False, needs_layout_passes=False),
        out_type=jax.ShapeDtypeStruct((32 * _NPAD,), jnp.float32),
        scratch_types=[
            pltpu.VMEM((_NPAD,), jnp.float32),
            pltpu.VMEM((_K,), jnp.int32),
        ],
    )
    def k(dst_hbm, out_hbm, cnt_v, dst_v):
        c = lax.axis_index("c")
        s = lax.axis_index("s")
        w = s * 2 + c
        zero16 = jnp.zeros((16,), jnp.float32)

        def zbody(i, _):
            cnt_v[pl.ds(i * 16, 16)] = zero16
            return 0
        lax.fori_loop(0, _NPAD // 16, zbody, 0)

        base = w * ept
        ones = jnp.ones((16,), jnp.float32)

        def chunk(j, _):
            pltpu.sync_copy(dst_hbm.at[pl.ds(base + j * _K, _K)], dst_v)

            def inner(i, _):
                idx = dst_v[pl.ds(i * 16, 16)]
                plsc.addupdate_scatter(cnt_v, [idx], ones)
                return 0
            lax.fori_loop(0, _K // 16, inner, 0)
            return 0
        lax.fori_loop(0, nch, chunk, 0)
        pltpu.sync_copy(cnt_v, out_hbm.at[pl.ds(w * _NPAD, _NPAD)])

    return k(dstp)


def _sc_invdeg(parts):
    stripe = _NPAD // 32
    mesh = plsc.VectorSubcoreMesh(core_axis_name="c", subcore_axis_name="s")

    @functools.partial(
        pl.kernel, mesh=mesh,
        compiler_params=pltpu.CompilerParams(
            use_tc_tiling_on_sc=False, needs_layout_passes=False),
        out_type=jax.ShapeDtypeStruct((_NPAD,), jnp.float32),
        scratch_types=[
            pltpu.VMEM((stripe,), jnp.float32),
            pltpu.VMEM((stripe,), jnp.float32),
        ],
    )
    def k(part_hbm, out_hbm, red_v, tmp_v):
        c = lax.axis_index("c")
        s = lax.axis_index("s")
        w = s * 2 + c
        o = w * stripe
        pltpu.sync_copy(part_hbm.at[pl.ds(o, stripe)], red_v)
        for p in range(1, 32):
            pltpu.sync_copy(part_hbm.at[pl.ds(p * _NPAD + o, stripe)], tmp_v)

            def abody(i, _):
                sl = pl.ds(i * 16, 16)
                red_v[sl] = red_v[sl] + tmp_v[sl]
                return 0
            lax.fori_loop(0, stripe // 16, abody, 0)

        one16 = jnp.full((16,), 1.0, jnp.float32)

        def ibody(i, _):
            sl = pl.ds(i * 16, 16)
            red_v[sl] = one16 / jnp.maximum(red_v[sl], one16)
            return 0
        lax.fori_loop(0, stripe // 16, ibody, 0)
        pltpu.sync_copy(red_v, out_hbm.at[pl.ds(o, stripe)])

    return k(parts)


def _sc_segsum(table_flat, srcp, dstp, zeros16, nhalves):
    npass = nhalves // 2
    ept = _EPAD // _NTILES
    nch = ept // _KS
    assert nch % 2 == 0
    mesh = plsc.VectorSubcoreMesh(core_axis_name="c", subcore_axis_name="s")

    @functools.partial(
        pl.kernel, mesh=mesh,
        compiler_params=pltpu.CompilerParams(
            use_tc_tiling_on_sc=False, needs_layout_passes=False),
        out_type=jax.ShapeDtypeStruct((nhalves, _NPAD, 16), jnp.float32),
        scratch_types=[
            [pltpu.VMEM((_KS,), jnp.int32)] * 2,
            [pltpu.VMEM((_KS,), jnp.int32)] * 2,
            [pltpu.VMEM((_KS,), jnp.int32)] * 2,
            [pltpu.VMEM((_KS, 16), jnp.float32)] * 2,
            pltpu.VMEM_SHARED((_NPAD, 16), jnp.float32),
            [pltpu.SemaphoreType.DMA] * 2,
        ],
    )
    def k(tab_hbm, src_hbm, dst_hbm, z_hbm, out_hbm,
          src_v, gidx_v, dst_v, rows_v, acc_sh, sem):
        c = lax.axis_index("c")
        s = lax.axis_index("s")
        r0 = s * _STRIPE
        base = s * ept

        def prep(j, b, h):
            e0 = base + j * _KS
            pltpu.sync_copy(src_hbm.at[pl.ds(e0, _KS)], src_v[b])
            pltpu.sync_copy(dst_hbm.at[pl.ds(e0, _KS)], dst_v[b])

            def xform(i, _):
                v = src_v[b][pl.ds(i * 16, 16)]
                gidx_v[b][pl.ds(i * 16, 16)] = v * nhalves + h
                return 0
            lax.fori_loop(0, _KS // 16, xform, 0)
            pltpu.async_copy(tab_hbm.at[gidx_v[b]], rows_v[b], sem[b])

        def consume(b):
            pltpu.make_async_copy(tab_hbm.at[gidx_v[b]], rows_v[b], sem[b]).wait()
            pltpu.sync_copy(rows_v[b], acc_sh.at[dst_v[b]], add=True)

        for p in range(npass):
            h = c * npass + p
            pltpu.sync_copy(z_hbm.at[pl.ds(r0, _STRIPE)], acc_sh.at[pl.ds(r0, _STRIPE)])
            plsc.subcore_barrier()

            prep(0, 0, h)

            def pair(jj, _, h=h):
                prep(2 * jj + 1, 1, h)
                consume(0)

                @pl.when(2 * jj + 2 < nch)
                def _():
                    prep(2 * jj + 2, 0, h)
                consume(1)
                return 0
            lax.fori_loop(0, nch // 2, pair, 0)
            plsc.subcore_barrier()
            pltpu.sync_copy(acc_sh.at[pl.ds(r0, _STRIPE)],
                            out_hbm.at[h, pl.ds(r0, _STRIPE)])
            plsc.subcore_barrier()

    return k(table_flat, srcp, dstp, zeros16)


def kernel(x, edge_index, eW1, eb1, eW2, eb2, eW3, eb3, dW1, db1, dW2, db2,
           dW3, db3, c1Wl, c1b, c1Wr, c2Wl, c2b, c2Wr, c3Wl, c3b, c3Wr, fcW, fcb):
    del dW1, db1, dW2, db2, dW3, db3

    eb1r = eb1.reshape(1, 32)
    eb2r = eb2.reshape(1, 32)
    eb3r = eb3.reshape(1, 32)
    c1br = c1b.reshape(1, 64)
    c2br = c2b.reshape(1, 64)
    c3br = c3b.reshape(1, 32)
    fcbr = fcb.reshape(1, 10)

    src = edge_index[0]
    dst = edge_index[1]
    srcp = jnp.concatenate([src, jnp.zeros((_EPAD - _E,), jnp.int32)])
    dstp = jnp.concatenate([dst, jnp.full((_EPAD - _E,), _NPAD - 1, jnp.int32)])
    zeros16 = jnp.zeros((_NPAD, 16), jnp.float32)

    inv = _sc_invdeg(_sc_degree(dstp)).reshape(_NPAD, 1)

    encoded = _tc_encoder(x, eW1, eb1r, eW2, eb2r, eW3, eb3r)

    agg1 = _sc_segsum(encoded.reshape(2 * _N, 16), srcp, dstp, zeros16, 2)
    h1 = _tc_combine(agg1, inv, encoded, c1Wl, c1Wr, c1br, 64)

    agg2 = _sc_segsum(h1.reshape(4 * _N, 16), srcp, dstp, zeros16, 4)
    p3, q3 = _tc_combine2(agg2, inv, h1, c2Wl, c2Wr, c2br, c3Wl, c3Wr)

    agg3 = _sc_segsum(p3.reshape(2 * _N, 16), srcp, dstp, zeros16, 2)
    return _tc_final(agg3, inv, q3, c3br, fcW, fcbr)


# speedup vs baseline: 8.0007x; 1.0573x over previous
import functools

import jax
import jax.numpy as jnp
from jax import lax
from jax.experimental import pallas as pl
from jax.experimental.pallas import tpu as pltpu
from jax.experimental.pallas import tpu_sc as plsc

_N = 100000
_E = 1600000
_NPAD = 102400
_EPAD = 1638400
_K = 1024
_KS = 512
_CPB = 8
_BN = 2000
_NTILES = 16
_STRIPE = _NPAD // _NTILES


def _full_spec(a):
    nd = a.ndim
    return pl.BlockSpec(a.shape, lambda i, _nd=nd: (0,) * _nd)


def _enc_body(x_ref, w1, b1, w2, b2, w3, b3, out_ref):
    h = jnp.maximum(jnp.dot(x_ref[...], w1[...], preferred_element_type=jnp.float32) + b1[...], 0.0)
    h = jnp.maximum(jnp.dot(h, w2[...], preferred_element_type=jnp.float32) + b2[...], 0.0)
    out_ref[...] = jnp.maximum(jnp.dot(h, w3[...], preferred_element_type=jnp.float32) + b3[...], 0.0)


def _tc_encoder(x, eW1, eb1, eW2, eb2, eW3, eb3):
    args = (x, eW1, eb1, eW2, eb2, eW3, eb3)
    return pl.pallas_call(
        _enc_body,
        grid=(_N // _BN,),
        in_specs=[pl.BlockSpec((_BN, 2), lambda i: (i, 0))] + [_full_spec(a) for a in args[1:]],
        out_specs=pl.BlockSpec((_BN, 32), lambda i: (i, 0)),
        out_shape=jax.ShapeDtypeStruct((_N, 32), jnp.float32),
    )(*args)


def _combine_body(agg_ref, inv_ref, prev_ref, wl, wr, b, out_ref):
    inv = inv_ref[...]
    a = agg_ref[...]
    s = jnp.concatenate([a[j] for j in range(a.shape[0])], axis=1) * inv
    out_ref[...] = jnp.maximum(
        jnp.dot(s, wl[...], preferred_element_type=jnp.float32)
        + jnp.dot(prev_ref[...], wr[...], preferred_element_type=jnp.float32)
        + b[...], 0.0)


def _tc_combine(agg, cnt, prev, wl, wr, b, out_dim):
    nh = agg.shape[0]
    args = (agg, cnt, prev, wl, wr, b)
    return pl.pallas_call(
        _combine_body,
        grid=(_N // _BN,),
        in_specs=[
            pl.BlockSpec((nh, _BN, 16), lambda i: (0, i, 0)),
            pl.BlockSpec((_BN, 1), lambda i: (i, 0)),
            pl.BlockSpec((_BN, prev.shape[1]), lambda i: (i, 0)),
            _full_spec(wl), _full_spec(wr), _full_spec(b),
        ],
        out_specs=pl.BlockSpec((_BN, out_dim), lambda i: (i, 0)),
        out_shape=jax.ShapeDtypeStruct((_N, out_dim), jnp.float32),
    )(*args)


def _combine2_body(agg_ref, inv_ref, prev_ref, wl, wr, b, w3l, w3r, p_ref, q_ref):
    inv = inv_ref[...]
    a = agg_ref[...]
    s = jnp.concatenate([a[j] for j in range(a.shape[0])], axis=1) * inv
    h2 = jnp.maximum(
        jnp.dot(s, wl[...], preferred_element_type=jnp.float32)
        + jnp.dot(prev_ref[...], wr[...], preferred_element_type=jnp.float32)
        + b[...], 0.0)
    p_ref[...] = jnp.dot(h2, w3l[...], preferred_element_type=jnp.float32)
    q_ref[...] = jnp.dot(h2, w3r[...], preferred_element_type=jnp.float32)


def _tc_combine2(agg, cnt, prev, wl, wr, b, w3l, w3r):
    args = (agg, cnt, prev, wl, wr, b, w3l, w3r)
    return pl.pallas_call(
        _combine2_body,
        grid=(_N // _BN,),
        in_specs=[
            pl.BlockSpec((4, _BN, 16), lambda i: (0, i, 0)),
            pl.BlockSpec((_BN, 1), lambda i: (i, 0)),
            pl.BlockSpec((_BN, 64), lambda i: (i, 0)),
            _full_spec(wl), _full_spec(wr), _full_spec(b),
            _full_spec(w3l), _full_spec(w3r),
        ],
        out_specs=[
            pl.BlockSpec((_BN, 32), lambda i: (i, 0)),
            pl.BlockSpec((_BN, 32), lambda i: (i, 0)),
        ],
        out_shape=[
            jax.ShapeDtypeStruct((_N, 32), jnp.float32),
            jax.ShapeDtypeStruct((_N, 32), jnp.float32),
        ],
    )(*args)


def _final_body(agg_ref, inv_ref, q_ref, b3, fcw, fcb, out_ref, acc_ref):
    i = pl.program_id(0)
    inv = inv_ref[...]
    a = agg_ref[...]
    h3 = jnp.maximum(
        jnp.concatenate([a[0], a[1]], axis=1) * inv + q_ref[...] + b3[...], 0.0)

    @pl.when(i == 0)
    def _():
        acc_ref[...] = jnp.zeros_like(acc_ref)

    acc_ref[...] = acc_ref[...] + jnp.sum(h3, axis=0, keepdims=True)

    @pl.when(i == pl.num_programs(0) - 1)
    def _():
        pooled = acc_ref[...] * (1.0 / _N)
        logits = jnp.dot(pooled, fcw[...], preferred_element_type=jnp.float32) + fcb[...]
        out_ref[...] = 1.0 / (1.0 + jnp.exp(-logits))


def _tc_final(agg, cnt, q, b3, fcW, fcb):
    args = (agg, cnt, q, b3, fcW, fcb)
    return pl.pallas_call(
        _final_body,
        grid=(_N // _BN,),
        in_specs=[
            pl.BlockSpec((2, _BN, 16), lambda i: (0, i, 0)),
            pl.BlockSpec((_BN, 1), lambda i: (i, 0)),
            pl.BlockSpec((_BN, 32), lambda i: (i, 0)),
            _full_spec(b3), _full_spec(fcW), _full_spec(fcb),
        ],
        out_specs=pl.BlockSpec((1, 10), lambda i: (0, 0)),
        out_shape=jax.ShapeDtypeStruct((1, 10), jnp.float32),
        scratch_shapes=[pltpu.VMEM((1, 32), jnp.float32)],
    )(*args)


def _sc_degree(dstp):
    ept = _EPAD // 32
    nch = ept // _K
    mesh = plsc.VectorSubcoreMesh(core_axis_name="c", subcore_axis_name="s")

    @functools.partial(
        pl.kernel, mesh=mesh,
        compiler_params=pltpu.CompilerParams(
            use_tc_tiling_on_sc=False, needs_layout_passes=False),
        out_type=jax.ShapeDtypeStruct((32 * _NPAD,), jnp.float32),
        scratch_types=[
            pltpu.VMEM((_NPAD,), jnp.float32),
            pltpu.VMEM((_K,), jnp.int32),
        ],
    )
    def k(dst_hbm, out_hbm, cnt_v, dst_v):
        c = lax.axis_index("c")
        s = lax.axis_index("s")
        w = s * 2 + c
        zero16 = jnp.zeros((16,), jnp.float32)

        def zbody(i, _):
            cnt_v[pl.ds(i * 16, 16)] = zero16
            return 0
        lax.fori_loop(0, _NPAD // 16, zbody, 0)

        base = w * ept
        ones = jnp.ones((16,), jnp.float32)

        def chunk(j, _):
            pltpu.sync_copy(dst_hbm.at[pl.ds(base + j * _K, _K)], dst_v)

            def inner(i, _):
                idx = dst_v[pl.ds(i * 16, 16)]
                plsc.addupdate_scatter(cnt_v, [idx], ones)
                return 0
            lax.fori_loop(0, _K // 16, inner, 0)
            return 0
        lax.fori_loop(0, nch, chunk, 0)
        pltpu.sync_copy(cnt_v, out_hbm.at[pl.ds(w * _NPAD, _NPAD)])

    return k(dstp)


def _sc_invdeg(parts):
    stripe = _NPAD // 32
    mesh = plsc.VectorSubcoreMesh(core_axis_name="c", subcore_axis_name="s")

    @functools.partial(
        pl.kernel, mesh=mesh,
        compiler_params=pltpu.CompilerParams(
            use_tc_tiling_on_sc=False, needs_layout_passes=False),
        out_type=jax.ShapeDtypeStruct((_NPAD,), jnp.float32),
        scratch_types=[
            pltpu.VMEM((stripe,), jnp.float32),
            pltpu.VMEM((stripe,), jnp.float32),
        ],
    )
    def k(part_hbm, out_hbm, red_v, tmp_v):
        c = lax.axis_index("c")
        s = lax.axis_index("s")
        w = s * 2 + c
        o = w * stripe
        pltpu.sync_copy(part_hbm.at[pl.ds(o, stripe)], red_v)
        for p in range(1, 32):
            pltpu.sync_copy(part_hbm.at[pl.ds(p * _NPAD + o, stripe)], tmp_v)

            def abody(i, _):
                sl = pl.ds(i * 16, 16)
                red_v[sl] = red_v[sl] + tmp_v[sl]
                return 0
            lax.fori_loop(0, stripe // 16, abody, 0)

        one16 = jnp.full((16,), 1.0, jnp.float32)

        def ibody(i, _):
            sl = pl.ds(i * 16, 16)
            red_v[sl] = one16 / jnp.maximum(red_v[sl], one16)
            return 0
        lax.fori_loop(0, stripe // 16, ibody, 0)
        pltpu.sync_copy(red_v, out_hbm.at[pl.ds(o, stripe)])

    return k(parts)


def _sc_segsum(table_flat, srcp, dstp, zeros16, nhalves):
    npass = nhalves // 2
    ept = _EPAD // _NTILES
    nch = ept // _KS
    nblk = nch // _CPB
    rows_per_tile = nch
    mesh = plsc.VectorSubcoreMesh(core_axis_name="c", subcore_axis_name="s")

    @functools.partial(
        pl.kernel, mesh=mesh,
        compiler_params=pltpu.CompilerParams(
            use_tc_tiling_on_sc=False, needs_layout_passes=False),
        out_type=jax.ShapeDtypeStruct((nhalves, _NPAD, 16), jnp.float32),
        scratch_types=[
            pltpu.VMEM((_CPB, _KS), jnp.int32),
            pltpu.VMEM((_CPB, _KS), jnp.int32),
            [pltpu.VMEM((_KS,), jnp.int32)] * 2,
            [pltpu.VMEM((_KS, 16), jnp.float32)] * 2,
            pltpu.VMEM_SHARED((_NPAD, 16), jnp.float32),
            [pltpu.SemaphoreType.DMA] * 2,
        ],
    )
    def k(tab_hbm, src_hbm, dst_hbm, z_hbm, out_hbm,
          src_blk, dst_blk, gidx_v, rows_v, acc_sh, sem):
        c = lax.axis_index("c")
        s = lax.axis_index("s")
        r0 = s * _STRIPE

        def prep(kk, b, h):
            def xform(i, _):
                v = src_blk[kk, pl.ds(i * 16, 16)]
                gidx_v[b][pl.ds(i * 16, 16)] = v * nhalves + h
                return 0
            lax.fori_loop(0, _KS // 16, xform, 0)
            pltpu.async_copy(tab_hbm.at[gidx_v[b]], rows_v[b], sem[b])

        def consume(kk, b):
            pltpu.make_async_copy(tab_hbm.at[gidx_v[b]], rows_v[b], sem[b]).wait()
            pltpu.sync_copy(rows_v[b], acc_sh.at[dst_blk.at[kk]], add=True)

        for p in range(npass):
            h = c * npass + p
            pltpu.sync_copy(z_hbm.at[pl.ds(r0, _STRIPE)], acc_sh.at[pl.ds(r0, _STRIPE)])
            plsc.subcore_barrier()

            def blk(bi, _, h=h):
                row0 = s * rows_per_tile + bi * _CPB
                pltpu.sync_copy(src_hbm.at[pl.ds(row0, _CPB)], src_blk)
                pltpu.sync_copy(dst_hbm.at[pl.ds(row0, _CPB)], dst_blk)
                prep(0, 0, h)
                for jj in range(_CPB // 2):
                    prep(2 * jj + 1, 1, h)
                    consume(2 * jj, 0)
                    if 2 * jj + 2 < _CPB:
                        prep(2 * jj + 2, 0, h)
                    consume(2 * jj + 1, 1)
                return 0
            lax.fori_loop(0, nblk, blk, 0)
            plsc.subcore_barrier()
            pltpu.sync_copy(acc_sh.at[pl.ds(r0, _STRIPE)],
                            out_hbm.at[h, pl.ds(r0, _STRIPE)])
            plsc.subcore_barrier()

    return k(table_flat, srcp, dstp, zeros16)


def kernel(x, edge_index, eW1, eb1, eW2, eb2, eW3, eb3, dW1, db1, dW2, db2,
           dW3, db3, c1Wl, c1b, c1Wr, c2Wl, c2b, c2Wr, c3Wl, c3b, c3Wr, fcW, fcb):
    del dW1, db1, dW2, db2, dW3, db3

    eb1r = eb1.reshape(1, 32)
    eb2r = eb2.reshape(1, 32)
    eb3r = eb3.reshape(1, 32)
    c1br = c1b.reshape(1, 64)
    c2br = c2b.reshape(1, 64)
    c3br = c3b.reshape(1, 32)
    fcbr = fcb.reshape(1, 10)

    src = edge_index[0]
    dst = edge_index[1]
    srcp = jnp.concatenate([src, jnp.zeros((_EPAD - _E,), jnp.int32)])
    dstp = jnp.concatenate([dst, jnp.full((_EPAD - _E,), _NPAD - 1, jnp.int32)])
    srcp2 = srcp.reshape(_EPAD // _KS, _KS)
    dstp2 = dstp.reshape(_EPAD // _KS, _KS)
    zeros16 = jnp.zeros((_NPAD, 16), jnp.float32)

    inv = _sc_invdeg(_sc_degree(dstp)).reshape(_NPAD, 1)

    encoded = _tc_encoder(x, eW1, eb1r, eW2, eb2r, eW3, eb3r)

    agg1 = _sc_segsum(encoded.reshape(2 * _N, 16), srcp2, dstp2, zeros16, 2)
    h1 = _tc_combine(agg1, inv, encoded, c1Wl, c1Wr, c1br, 64)

    agg2 = _sc_segsum(h1.reshape(4 * _N, 16), srcp2, dstp2, zeros16, 4)
    p3, q3 = _tc_combine2(agg2, inv, h1, c2Wl, c2Wr, c2br, c3Wl, c3Wr)

    agg3 = _sc_segsum(p3.reshape(2 * _N, 16), srcp2, dstp2, zeros16, 2)
    return _tc_final(agg3, inv, q3, c3br, fcW, fcbr)


# speedup vs baseline: 8.1045x; 1.0130x over previous
import functools

import jax
import jax.numpy as jnp
from jax import lax
from jax.experimental import pallas as pl
from jax.experimental.pallas import tpu as pltpu
from jax.experimental.pallas import tpu_sc as plsc

_N = 100000
_E = 1600000
_NPAD = 102400
_EPAD = 1638400
_K = 1024
_KS = 512
_CPB = 8
_BN = 2000
_NTILES = 16
_STRIPE = _NPAD // _NTILES


def _full_spec(a):
    nd = a.ndim
    return pl.BlockSpec(a.shape, lambda i, _nd=nd: (0,) * _nd)


def _enc_body(x_ref, w1, b1, w2, b2, w3, b3, out_ref):
    h = jnp.maximum(jnp.dot(x_ref[...], w1[...], preferred_element_type=jnp.float32) + b1[...], 0.0)
    h = jnp.maximum(jnp.dot(h, w2[...], preferred_element_type=jnp.float32) + b2[...], 0.0)
    out_ref[...] = jnp.maximum(jnp.dot(h, w3[...], preferred_element_type=jnp.float32) + b3[...], 0.0)


def _tc_encoder(x, eW1, eb1, eW2, eb2, eW3, eb3):
    args = (x, eW1, eb1, eW2, eb2, eW3, eb3)
    return pl.pallas_call(
        _enc_body,
        grid=(_N // _BN,),
        in_specs=[pl.BlockSpec((_BN, 2), lambda i: (i, 0))] + [_full_spec(a) for a in args[1:]],
        out_specs=pl.BlockSpec((_BN, 32), lambda i: (i, 0)),
        out_shape=jax.ShapeDtypeStruct((_N, 32), jnp.float32),
    )(*args)


def _combine_body(agg_ref, inv_ref, prev_ref, wl, wr, b, out_ref):
    inv = inv_ref[...]
    a = agg_ref[...]
    s = jnp.concatenate([a[j] for j in range(a.shape[0])], axis=1) * inv
    out_ref[...] = jnp.maximum(
        jnp.dot(s, wl[...], preferred_element_type=jnp.float32)
        + jnp.dot(prev_ref[...], wr[...], preferred_element_type=jnp.float32)
        + b[...], 0.0)


def _tc_combine(agg, cnt, prev, wl, wr, b, out_dim):
    nh = agg.shape[0]
    args = (agg, cnt, prev, wl, wr, b)
    return pl.pallas_call(
        _combine_body,
        grid=(_N // _BN,),
        in_specs=[
            pl.BlockSpec((nh, _BN, 16), lambda i: (0, i, 0)),
            pl.BlockSpec((_BN, 1), lambda i: (i, 0)),
            pl.BlockSpec((_BN, prev.shape[1]), lambda i: (i, 0)),
            _full_spec(wl), _full_spec(wr), _full_spec(b),
        ],
        out_specs=pl.BlockSpec((_BN, out_dim), lambda i: (i, 0)),
        out_shape=jax.ShapeDtypeStruct((_N, out_dim), jnp.float32),
    )(*args)


def _combine2_body(agg_ref, inv_ref, prev_ref, wl, wr, b, w3l, w3r, p_ref, q_ref):
    inv = inv_ref[...]
    a = agg_ref[...]
    s = jnp.concatenate([a[j] for j in range(a.shape[0])], axis=1) * inv
    h2 = jnp.maximum(
        jnp.dot(s, wl[...], preferred_element_type=jnp.float32)
        + jnp.dot(prev_ref[...], wr[...], preferred_element_type=jnp.float32)
        + b[...], 0.0)
    p_ref[...] = jnp.dot(h2, w3l[...], preferred_element_type=jnp.float32)
    q_ref[...] = jnp.dot(h2, w3r[...], preferred_element_type=jnp.float32)


def _tc_combine2(agg, cnt, prev, wl, wr, b, w3l, w3r):
    args = (agg, cnt, prev, wl, wr, b, w3l, w3r)
    return pl.pallas_call(
        _combine2_body,
        grid=(_N // _BN,),
        in_specs=[
            pl.BlockSpec((4, _BN, 16), lambda i: (0, i, 0)),
            pl.BlockSpec((_BN, 1), lambda i: (i, 0)),
            pl.BlockSpec((_BN, 64), lambda i: (i, 0)),
            _full_spec(wl), _full_spec(wr), _full_spec(b),
            _full_spec(w3l), _full_spec(w3r),
        ],
        out_specs=[
            pl.BlockSpec((_BN, 32), lambda i: (i, 0)),
            pl.BlockSpec((_BN, 32), lambda i: (i, 0)),
        ],
        out_shape=[
            jax.ShapeDtypeStruct((_N, 32), jnp.float32),
            jax.ShapeDtypeStruct((_N, 32), jnp.float32),
        ],
    )(*args)


def _final_body(agg_ref, inv_ref, q_ref, b3, fcw, fcb, out_ref, acc_ref):
    i = pl.program_id(0)
    inv = inv_ref[...]
    a = agg_ref[...]
    h3 = jnp.maximum(
        jnp.concatenate([a[0], a[1]], axis=1) * inv + q_ref[...] + b3[...], 0.0)

    @pl.when(i == 0)
    def _():
        acc_ref[...] = jnp.zeros_like(acc_ref)

    acc_ref[...] = acc_ref[...] + jnp.sum(h3, axis=0, keepdims=True)

    @pl.when(i == pl.num_programs(0) - 1)
    def _():
        pooled = acc_ref[...] * (1.0 / _N)
        logits = jnp.dot(pooled, fcw[...], preferred_element_type=jnp.float32) + fcb[...]
        out_ref[...] = 1.0 / (1.0 + jnp.exp(-logits))


def _tc_final(agg, cnt, q, b3, fcW, fcb):
    args = (agg, cnt, q, b3, fcW, fcb)
    return pl.pallas_call(
        _final_body,
        grid=(_N // _BN,),
        in_specs=[
            pl.BlockSpec((2, _BN, 16), lambda i: (0, i, 0)),
            pl.BlockSpec((_BN, 1), lambda i: (i, 0)),
            pl.BlockSpec((_BN, 32), lambda i: (i, 0)),
            _full_spec(b3), _full_spec(fcW), _full_spec(fcb),
        ],
        out_specs=pl.BlockSpec((1, 10), lambda i: (0, 0)),
        out_shape=jax.ShapeDtypeStruct((1, 10), jnp.float32),
        scratch_shapes=[pltpu.VMEM((1, 32), jnp.float32)],
    )(*args)


def _sc_degree(dstp):
    ept = _EPAD // 32
    nch = ept // _K
    mesh = plsc.VectorSubcoreMesh(core_axis_name="c", subcore_axis_name="s")

    @functools.partial(
        pl.kernel, mesh=mesh,
        compiler_params=pltpu.CompilerParams(
            use_tc_tiling_on_sc=False, needs_layout_passes=False),
        out_type=jax.ShapeDtypeStruct((32 * _NPAD,), jnp.float32),
        scratch_types=[
            pltpu.VMEM((_NPAD,), jnp.float32),
            pltpu.VMEM((_K,), jnp.int32),
        ],
    )
    def k(dst_hbm, out_hbm, cnt_v, dst_v):
        c = lax.axis_index("c")
        s = lax.axis_index("s")
        w = s * 2 + c
        zero16 = jnp.zeros((16,), jnp.float32)

        def zbody(i, _):
            cnt_v[pl.ds(i * 16, 16)] = zero16
            return 0
        lax.fori_loop(0, _NPAD // 16, zbody, 0)

        base = w * ept
        ones = jnp.ones((16,), jnp.float32)

        def chunk(j, _):
            pltpu.sync_copy(dst_hbm.at[pl.ds(base + j * _K, _K)], dst_v)

            def inner(i, _):
                idx = dst_v[pl.ds(i * 16, 16)]
                plsc.addupdate_scatter(cnt_v, [idx], ones)
                return 0
            lax.fori_loop(0, _K // 16, inner, 0)
            return 0
        lax.fori_loop(0, nch, chunk, 0)
        pltpu.sync_copy(cnt_v, out_hbm.at[pl.ds(w * _NPAD, _NPAD)])

    return k(dstp)


def _sc_invdeg(parts):
    stripe = _NPAD // 32
    mesh = plsc.VectorSubcoreMesh(core_axis_name="c", subcore_axis_name="s")

    @functools.partial(
        pl.kernel, mesh=mesh,
        compiler_params=pltpu.CompilerParams(
            use_tc_tiling_on_sc=False, needs_layout_passes=False),
        out_type=jax.ShapeDtypeStruct((_NPAD,), jnp.float32),
        scratch_types=[
            pltpu.VMEM((stripe,), jnp.float32),
            pltpu.VMEM((stripe,), jnp.float32),
        ],
    )
    def k(part_hbm, out_hbm, red_v, tmp_v):
        c = lax.axis_index("c")
        s = lax.axis_index("s")
        w = s * 2 + c
        o = w * stripe
        pltpu.sync_copy(part_hbm.at[pl.ds(o, stripe)], red_v)
        for p in range(1, 32):
            pltpu.sync_copy(part_hbm.at[pl.ds(p * _NPAD + o, stripe)], tmp_v)

            def abody(i, _):
                sl = pl.ds(i * 16, 16)
                red_v[sl] = red_v[sl] + tmp_v[sl]
                return 0
            lax.fori_loop(0, stripe // 16, abody, 0)

        one16 = jnp.full((16,), 1.0, jnp.float32)

        def ibody(i, _):
            sl = pl.ds(i * 16, 16)
            red_v[sl] = one16 / jnp.maximum(red_v[sl], one16)
            return 0
        lax.fori_loop(0, stripe // 16, ibody, 0)
        pltpu.sync_copy(red_v, out_hbm.at[pl.ds(o, stripe)])

    return k(parts)


def _sc_segsum(table_flat, srcp, dstp, zeros16, nhalves):
    npass = nhalves // 2
    ept = _EPAD // _NTILES
    nch = ept // _KS
    nblk = nch // _CPB
    rows_per_tile = nch
    mesh = plsc.VectorSubcoreMesh(core_axis_name="c", subcore_axis_name="s")

    @functools.partial(
        pl.kernel, mesh=mesh,
        compiler_params=pltpu.CompilerParams(
            use_tc_tiling_on_sc=False, needs_layout_passes=False),
        out_type=jax.ShapeDtypeStruct((nhalves, _NPAD, 16), jnp.float32),
        scratch_types=[
            pltpu.VMEM((_CPB, _KS), jnp.int32),
            pltpu.VMEM((_CPB, _KS), jnp.int32),
            [pltpu.VMEM((_KS,), jnp.int32)] * 2,
            [pltpu.VMEM((_KS, 16), jnp.float32)] * 2,
            pltpu.VMEM_SHARED((_NPAD, 16), jnp.float32),
            [pltpu.SemaphoreType.DMA] * 2,
            [pltpu.SemaphoreType.DMA] * 2,
        ],
    )
    def k(tab_hbm, src_hbm, dst_hbm, z_hbm, out_hbm,
          src_blk, dst_blk, gidx_v, rows_v, acc_sh, gsem, ssem):
        c = lax.axis_index("c")
        s = lax.axis_index("s")
        r0 = s * _STRIPE

        def wait_gather(b):
            pltpu.make_async_copy(tab_hbm.at[gidx_v[b]], rows_v[b], gsem[b]).wait()

        def wait_scatter(kk, b):
            pltpu.make_async_copy(rows_v[b], acc_sh.at[dst_blk.at[kk]], ssem[b]).wait()

        for p in range(npass):
            h = c * npass + p
            pltpu.sync_copy(z_hbm.at[pl.ds(r0, _STRIPE)], acc_sh.at[pl.ds(r0, _STRIPE)])
            plsc.subcore_barrier()

            def blk(bi, _, h=h):
                row0 = s * rows_per_tile + bi * _CPB
                pltpu.sync_copy(src_hbm.at[pl.ds(row0, _CPB)], src_blk)
                pltpu.sync_copy(dst_hbm.at[pl.ds(row0, _CPB)], dst_blk)
                for kk in range(_CPB):
                    b = kk % 2

                    def xform(i, _, kk=kk, b=b):
                        v = src_blk[kk, pl.ds(i * 16, 16)]
                        gidx_v[b][pl.ds(i * 16, 16)] = v * nhalves + h
                        return 0
                    lax.fori_loop(0, _KS // 16, xform, 0)
                    if kk >= 2:
                        wait_scatter(kk - 2, b)
                    pltpu.async_copy(tab_hbm.at[gidx_v[b]], rows_v[b], gsem[b])
                    if kk >= 1:
                        wait_gather(1 - b)
                        pltpu.async_copy(rows_v[1 - b], acc_sh.at[dst_blk.at[kk - 1]],
                                         ssem[1 - b], add=True)
                wait_gather(1)
                pltpu.async_copy(rows_v[1], acc_sh.at[dst_blk.at[_CPB - 1]],
                                 ssem[1], add=True)
                wait_scatter(_CPB - 2, 0)
                wait_scatter(_CPB - 1, 1)
                return 0
            lax.fori_loop(0, nblk, blk, 0)
            plsc.subcore_barrier()
            pltpu.sync_copy(acc_sh.at[pl.ds(r0, _STRIPE)],
                            out_hbm.at[h, pl.ds(r0, _STRIPE)])
            plsc.subcore_barrier()

    return k(table_flat, srcp, dstp, zeros16)


def kernel(x, edge_index, eW1, eb1, eW2, eb2, eW3, eb3, dW1, db1, dW2, db2,
           dW3, db3, c1Wl, c1b, c1Wr, c2Wl, c2b, c2Wr, c3Wl, c3b, c3Wr, fcW, fcb):
    del dW1, db1, dW2, db2, dW3, db3

    eb1r = eb1.reshape(1, 32)
    eb2r = eb2.reshape(1, 32)
    eb3r = eb3.reshape(1, 32)
    c1br = c1b.reshape(1, 64)
    c2br = c2b.reshape(1, 64)
    c3br = c3b.reshape(1, 32)
    fcbr = fcb.reshape(1, 10)

    src = edge_index[0]
    dst = edge_index[1]
    srcp = jnp.concatenate([src, jnp.zeros((_EPAD - _E,), jnp.int32)])
    dstp = jnp.concatenate([dst, jnp.full((_EPAD - _E,), _NPAD - 1, jnp.int32)])
    srcp2 = srcp.reshape(_EPAD // _KS, _KS)
    dstp2 = dstp.reshape(_EPAD // _KS, _KS)
    zeros16 = jnp.zeros((_NPAD, 16), jnp.float32)

    inv = _sc_invdeg(_sc_degree(dstp)).reshape(_NPAD, 1)

    encoded = _tc_encoder(x, eW1, eb1r, eW2, eb2r, eW3, eb3r)

    agg1 = _sc_segsum(encoded.reshape(2 * _N, 16), srcp2, dstp2, zeros16, 2)
    h1 = _tc_combine(agg1, inv, encoded, c1Wl, c1Wr, c1br, 64)

    agg2 = _sc_segsum(h1.reshape(4 * _N, 16), srcp2, dstp2, zeros16, 4)
    p3, q3 = _tc_combine2(agg2, inv, h1, c2Wl, c2Wr, c2br, c3Wl, c3Wr)

    agg3 = _sc_segsum(p3.reshape(2 * _N, 16), srcp2, dstp2, zeros16, 2)
    return _tc_final(agg3, inv, q3, c3br, fcW, fcbr)
